# Initial kernel scaffold; baseline (speedup 1.0000x reference)
import jax
import jax.numpy as jnp
from jax.experimental import pallas as pl


def kernel(edge_index, edge_type, dynamic_emb, emb_rel, w_self, time_gate_weight, time_gate_bias, gru_w_ih, gru_w_hh, gru_b_ih, gru_b_hh):
    raise NotImplementedError("write your pallas kernel here")


# speedup vs baseline: 2.5158x; 2.5158x over previous
import functools
import jax
import jax.numpy as jnp
from jax import lax
from jax.experimental import pallas as pl
from jax.experimental.pallas import tpu as pltpu
from jax.experimental.pallas import tpu_sc as plsc

N_ENTS = 10000
H = 128
R2 = 400
RP = 512
CW = 1024
E = 320000
SEQ_LEN = 3

NC = 2
NS = 16
NW = NC * NS

NEG_SLOPE = 0.22916666666666666


_B_REGIONS = 8
_ROWS_PER_REGION = N_ENTS // _B_REGIONS
_REGION_ELEMS = _ROWS_PER_REGION * CW
_ZCHUNK = 2000
_NZCHUNK = _REGION_ELEMS // _ZCHUNK
_BE = 80
_ECHUNKS = E // NS // _BE


def _build_body(dst_hbm, src_hbm, et_hbm, out_hbm,
                region, zbuf, dst_v, src_v, et_v, i1_v, v1_v, i2_v, v2_v):
    c = lax.axis_index("c")
    s = lax.axis_index("s")

    z16 = jnp.zeros((16,), jnp.float32)
    for u in range(_ZCHUNK // 16):
        zbuf[pl.ds(u * 16, 16)] = z16

    def round_body(j, _):
        q = j * NC + c
        base = q * _ROWS_PER_REGION

        def zero_body(k, _):
            off = (k * NS + s) * _ZCHUNK
            pltpu.sync_copy(zbuf, region.at[pl.ds(off, _ZCHUNK)])
            return 0
        lax.fori_loop(0, _NZCHUNK // NS, zero_body, 0)
        plsc.subcore_barrier()

        def chunk_body(i, _):
            e0 = s * (E // NS) + i * _BE
            pltpu.sync_copy(dst_hbm.at[pl.ds(e0, _BE)], dst_v)
            pltpu.sync_copy(src_hbm.at[pl.ds(e0, _BE)], src_v)
            pltpu.sync_copy(et_hbm.at[pl.ds(e0, _BE)], et_v)
            for u in range(_BE // 16):
                sl = pl.ds(u * 16, 16)
                d16 = dst_v[sl]
                s16 = src_v[sl]
                e16 = et_v[sl]
                rd = d16 - base
                ind = (rd >= 0) & (rd < _ROWS_PER_REGION)
                i1_v[sl] = jnp.where(ind, rd * CW + e16, 0)
                v1_v[sl] = jnp.where(ind, 1.0, 0.0)
                rs = s16 - base
                ins = (rs >= 0) & (rs < _ROWS_PER_REGION)
                i2_v[sl] = jnp.where(ins, rs * CW + 512 + e16, 0)
                v2_v[sl] = jnp.where(ins, 1.0, 0.0)
            pltpu.sync_copy(v1_v, region.at[i1_v], add=True)
            pltpu.sync_copy(v2_v, region.at[i2_v], add=True)
            return 0
        lax.fori_loop(0, _ECHUNKS, chunk_body, 0)
        plsc.subcore_barrier()

        def out_body(k, _):
            off = (k * NS + s) * _ZCHUNK
            pltpu.sync_copy(region.at[pl.ds(off, _ZCHUNK)], zbuf)
            pltpu.sync_copy(zbuf,
                            out_hbm.at[pl.ds(q * _REGION_ELEMS + off, _ZCHUNK)])
            return 0
        lax.fori_loop(0, _NZCHUNK // NS, out_body, 0)
        plsc.subcore_barrier()

        for u in range(_ZCHUNK // 16):
            zbuf[pl.ds(u * 16, 16)] = z16
        return 0

    lax.fori_loop(0, _B_REGIONS // NC, round_body, 0)


def _sc_build_counts(dst, src, et):
    mesh = plsc.VectorSubcoreMesh(core_axis_name="c", subcore_axis_name="s", num_cores=NC, num_subcores=NS)
    k = pl.kernel(
        _build_body,
        out_type=jax.ShapeDtypeStruct((N_ENTS * CW,), jnp.float32),
        mesh=mesh,
        scratch_types=[
            pltpu.VMEM_SHARED((_REGION_ELEMS,), jnp.float32),
            pltpu.VMEM((_ZCHUNK,), jnp.float32),
            pltpu.VMEM((_BE,), jnp.int32),
            pltpu.VMEM((_BE,), jnp.int32),
            pltpu.VMEM((_BE,), jnp.int32),
            pltpu.VMEM((_BE,), jnp.int32),
            pltpu.VMEM((_BE,), jnp.float32),
            pltpu.VMEM((_BE,), jnp.int32),
            pltpu.VMEM((_BE,), jnp.float32),
        ],
    )
    return k(dst, src, et)


_EB = 80
_NCHUNK = E // NW // _EB
_ZROWS = N_ENTS // NS


def _edge_body(h_hbm, src_hbm, dst_hbm, out_hbm,
               agg, src_v, dst_v, rows_v, sem):
    c = lax.axis_index("c")
    s = lax.axis_index("s")
    wid = c * NS + s

    z16 = jnp.zeros((16,), jnp.float32)
    for rr in range(_EB):
        for cc in range(H // 16):
            rows_v[rr, pl.ds(cc * 16, 16)] = z16
    base = s * 624

    @pl.when(s < NS - 1)
    def _():
        for p in range(7):
            pltpu.sync_copy(rows_v, agg.at[pl.ds(base + p * _EB, _EB), :])
        pltpu.sync_copy(rows_v.at[pl.ds(0, 64), :],
                        agg.at[pl.ds(base + 560, 64), :])

    @pl.when(s == NS - 1)
    def _():
        for p in range(8):
            pltpu.sync_copy(rows_v, agg.at[pl.ds(9360 + p * _EB, _EB), :])
    plsc.subcore_barrier()

    def chunk_body(i, _):
        e0 = wid * (E // NW) + i * _EB
        pltpu.sync_copy(src_hbm.at[pl.ds(e0, _EB)], src_v)
        pltpu.sync_copy(dst_hbm.at[pl.ds(e0, _EB)], dst_v)
        pltpu.async_copy(h_hbm.at[src_v], rows_v, sem).wait()
        pltpu.sync_copy(rows_v, agg.at[dst_v], add=True)
        return 0
    lax.fori_loop(0, _NCHUNK, chunk_body, 0)
    plsc.subcore_barrier()

    row0 = c * N_ENTS + base

    @pl.when(s < NS - 1)
    def _():
        for p in range(7):
            pltpu.sync_copy(agg.at[pl.ds(base + p * _EB, _EB), :], rows_v)
            pltpu.sync_copy(rows_v, out_hbm.at[pl.ds(row0 + p * _EB, _EB), :])
        pltpu.sync_copy(agg.at[pl.ds(base + 560, 64), :],
                        rows_v.at[pl.ds(0, 64), :])
        pltpu.sync_copy(rows_v.at[pl.ds(0, 64), :],
                        out_hbm.at[pl.ds(row0 + 560, 64), :])

    @pl.when(s == NS - 1)
    def _():
        for p in range(8):
            pltpu.sync_copy(agg.at[pl.ds(9360 + p * _EB, _EB), :], rows_v)
            pltpu.sync_copy(
                rows_v,
                out_hbm.at[pl.ds(c * N_ENTS + 9360 + p * _EB, _EB), :])


def _sc_edge_pass(h, src, dst):
    mesh = plsc.VectorSubcoreMesh(core_axis_name="c", subcore_axis_name="s", num_cores=NC, num_subcores=NS)
    k = pl.kernel(
        _edge_body,
        out_type=jax.ShapeDtypeStruct((NC * N_ENTS, H), jnp.float32),
        mesh=mesh,
        scratch_types=[
            pltpu.VMEM_SHARED((N_ENTS, H), jnp.float32),
            pltpu.VMEM((_EB,), jnp.int32),
            pltpu.VMEM((_EB,), jnp.int32),
            pltpu.VMEM((_EB, H), jnp.float32),
            pltpu.SemaphoreType.DMA,
        ],
    )
    return k(h, src, dst)


def _l2norm(x):
    n = jnp.sqrt(jnp.sum(x * x, axis=-1, keepdims=True))
    return x / jnp.maximum(n, 1e-12)


def _prologue_body(x_ref, o_ref):
    o_ref[...] = _l2norm(x_ref[...])


def _tc_prologue(x):
    return pl.pallas_call(
        _prologue_body,
        out_shape=jax.ShapeDtypeStruct((N_ENTS, H), jnp.float32),
    )(x)


_NB = 5
_RB = N_ENTS // _NB


def _rel_body(sp_ref, h_ref, r_ref, er_ref, wih_ref, whh_ref, bih_ref,
              bhh_ref, rn_ref, sums_acc, cnt_acc):
    i = pl.program_id(0)

    @pl.when(i == 0)
    def _():
        sums_acc[...] = jnp.zeros_like(sums_acc)
        cnt_acc[...] = jnp.zeros_like(cnt_acc)

    sp = sp_ref[...]
    h = h_ref[...]
    sums_acc[...] += jax.lax.dot_general(
        sp, h, (((0,), (0,)), ((), ())), preferred_element_type=jnp.float32)
    cnt_acc[0, :] += jnp.sum(sp, axis=0)

    @pl.when(i == _NB - 1)
    def _():
        r = r_ref[...]
        cnt = cnt_acc[0, :]
        x_mean = sums_acc[...] / jnp.maximum(cnt, 1.0)[:, None]
        x_cat = jnp.concatenate([x_mean, er_ref[...]], axis=1)
        gi = jnp.dot(x_cat, wih_ref[...],
                     preferred_element_type=jnp.float32) + bih_ref[...]
        gh = jnp.dot(r, whh_ref[...],
                     preferred_element_type=jnp.float32) + bhh_ref[...]
        rg = jax.nn.sigmoid(gi[:, :H] + gh[:, :H])
        z = jax.nn.sigmoid(gi[:, H:2 * H] + gh[:, H:2 * H])
        n = jnp.tanh(gi[:, 2 * H:] + rg * gh[:, 2 * H:])
        rn_ref[...] = (1.0 - z) * n + z * r


def _tc_relation(sp, h, r, er_pad, wih_t, whh_t, bih, bhh):
    return pl.pallas_call(
        _rel_body,
        grid=(_NB,),
        in_specs=[
            pl.BlockSpec((_RB, 512), lambda i: (i, 0)),
            pl.BlockSpec((_RB, H), lambda i: (i, 0)),
            pl.BlockSpec((RP, H), lambda i: (0, 0)),
            pl.BlockSpec((RP, H), lambda i: (0, 0)),
            pl.BlockSpec((2 * H, 3 * H), lambda i: (0, 0)),
            pl.BlockSpec((H, 3 * H), lambda i: (0, 0)),
            pl.BlockSpec((1, 3 * H), lambda i: (0, 0)),
            pl.BlockSpec((1, 3 * H), lambda i: (0, 0)),
        ],
        out_specs=pl.BlockSpec((RP, H), lambda i: (0, 0)),
        out_shape=jax.ShapeDtypeStruct((RP, H), jnp.float32),
        scratch_shapes=[
            pltpu.VMEM((RP, H), jnp.float32),
            pltpu.VMEM((1, 512), jnp.float32),
        ],
    )(sp, h, r, er_pad, wih_t, whh_t, bih, bhh)


def _node_body(c_ref, a0_ref, a1_ref, h_ref, rn_ref, ws_ref, tgw_ref,
               tgb_ref, o_ref):
    cm = c_ref[...]
    h = h_ref[...]
    deg = jnp.sum(cm, axis=1)
    inv_deg = 1.0 / jnp.maximum(deg, 1.0)
    ragg = jnp.dot(cm, rn_ref[...], preferred_element_type=jnp.float32)
    agg = (a0_ref[...] + a1_ref[...] - ragg) * inv_deg[:, None]
    loop = jnp.dot(h, ws_ref[...], preferred_element_type=jnp.float32)
    x = agg + loop
    x = jnp.where(x >= 0, x, x * NEG_SLOPE)
    x = _l2norm(x)
    gate = jax.nn.sigmoid(
        jnp.dot(x, tgw_ref[...], preferred_element_type=jnp.float32)
        + tgb_ref[...])
    o_ref[...] = gate * x + (1.0 - gate) * h


def _tc_node(cm, a0, a1, h, rn, w_self, tgw, tgb):
    return pl.pallas_call(
        _node_body,
        grid=(_NB,),
        in_specs=[
            pl.BlockSpec((_RB, 512), lambda i: (i, 0)),
            pl.BlockSpec((_RB, H), lambda i: (i, 0)),
            pl.BlockSpec((_RB, H), lambda i: (i, 0)),
            pl.BlockSpec((_RB, H), lambda i: (i, 0)),
            pl.BlockSpec((RP, H), lambda i: (0, 0)),
            pl.BlockSpec((H, H), lambda i: (0, 0)),
            pl.BlockSpec((H, H), lambda i: (0, 0)),
            pl.BlockSpec((1, H), lambda i: (0, 0)),
        ],
        out_specs=pl.BlockSpec((_RB, H), lambda i: (i, 0)),
        out_shape=jax.ShapeDtypeStruct((N_ENTS, H), jnp.float32),
    )(cm, a0, a1, h, rn, w_self, tgw, tgb)


def kernel(edge_index, edge_type, dynamic_emb, emb_rel, w_self,
           time_gate_weight, time_gate_bias, gru_w_ih, gru_w_hh,
           gru_b_ih, gru_b_hh):
    src = edge_index[0]
    dst = edge_index[1]

    counts = _sc_build_counts(dst, src, edge_type)
    counts = counts.reshape(N_ENTS, CW)
    cm = counts[:, :512]
    sp = counts[:, 512:]

    er_pad = jnp.zeros((RP, H), jnp.float32).at[:R2].set(emb_rel)
    wih_t = gru_w_ih.T
    whh_t = gru_w_hh.T
    bih = gru_b_ih.reshape(1, 3 * H)
    bhh = gru_b_hh.reshape(1, 3 * H)
    tgb = time_gate_bias.reshape(1, H)

    h = _tc_prologue(dynamic_emb)
    r = er_pad
    for _ in range(SEQ_LEN):
        aggp = _sc_edge_pass(h, src, dst)
        rn = _tc_relation(sp, h, r, er_pad, wih_t, whh_t, bih, bhh)
        h = _tc_node(cm, aggp[:N_ENTS], aggp[N_ENTS:], h, rn, w_self,
                     time_gate_weight, tgb)
        r = rn
    return h
